# baseline (speedup 1.0000x reference)
import functools

import jax
import jax.numpy as jnp
from jax import lax
from jax.experimental import pallas as pl
from jax.experimental.pallas import tpu as pltpu
from jax.experimental.pallas import tpu_sc as plsc

E = 16
TOP_K = 2
CONF_THRESH = 0.7
TILE = 1024

NUM_CORES = 2
NUM_SUBCORES = 16
NUM_WORKERS = NUM_CORES * NUM_SUBCORES
LANES = 16


def _probs_body(x_ref, w_ref, probs_ref):
    logits = jnp.dot(x_ref[...], w_ref[...], preferred_element_type=jnp.float32)
    lt = logits.T
    m = jnp.max(lt, axis=0, keepdims=True)
    e = jnp.exp(lt - m)
    z = jnp.sum(e, axis=0, keepdims=True)
    probs_ref[...] = e / z


def _tc_probs(x2, W_g):
    N, D = x2.shape
    grid = (N // TILE,)
    return pl.pallas_call(
        _probs_body,
        grid=grid,
        in_specs=[
            pl.BlockSpec((TILE, D), lambda i: (i, 0)),
            pl.BlockSpec((D, E), lambda i: (0, 0)),
        ],
        out_specs=pl.BlockSpec((E, TILE), lambda i: (0, i)),
        out_shape=jax.ShapeDtypeStruct((E, N), jnp.float32),
    )(x2, W_g)


def _sc_topk_mask(probs_t):
    _, N = probs_t.shape
    per_w = N // NUM_WORKERS
    groups = per_w // LANES
    mesh = plsc.VectorSubcoreMesh(core_axis_name="c", subcore_axis_name="s")

    @functools.partial(
        pl.kernel,
        out_type=[
            jax.ShapeDtypeStruct((N,), jnp.float32),
            jax.ShapeDtypeStruct((N,), jnp.float32),
            jax.ShapeDtypeStruct((N,), jnp.int32),
            jax.ShapeDtypeStruct((N,), jnp.int32),
        ],
        mesh=mesh,
        scratch_types=[
            pltpu.VMEM((E, per_w), jnp.float32),
            pltpu.VMEM((per_w,), jnp.float32),
            pltpu.VMEM((per_w,), jnp.float32),
            pltpu.VMEM((per_w,), jnp.int32),
            pltpu.VMEM((per_w,), jnp.int32),
        ],
    )
    def sc_kernel(p_hbm, w1_hbm, w2_hbm, i1_hbm, i2_hbm,
                  p_v, w1_v, w2_v, i1_v, i2_v):
        wid = lax.axis_index("s") * NUM_CORES + lax.axis_index("c")
        base = wid * per_w
        pltpu.sync_copy(p_hbm.at[:, pl.ds(base, per_w)], p_v)

        neg1f = jnp.full((LANES,), -1.0, jnp.float32)
        neg1i = jnp.full((LANES,), -1, jnp.int32)
        bigi = jnp.full((LANES,), E, jnp.int32)
        thresh = jnp.full((LANES,), CONF_THRESH, jnp.float32)

        def group(g, carry):
            off = g * LANES
            p = [p_v[e, pl.ds(off, LANES)] for e in range(E)]
            m1 = p[0]
            for e in range(1, E):
                m1 = jnp.maximum(m1, p[e])
            i1 = bigi
            for e in range(E):
                e_s = jnp.full((LANES,), e, jnp.int32)
                i1 = jnp.minimum(i1, jnp.where(p[e] == m1, e_s, bigi))
            m2 = neg1f
            for e in range(E):
                e_s = jnp.full((LANES,), e, jnp.int32)
                m2 = jnp.maximum(m2, jnp.where(e_s == i1, neg1f, p[e]))
            i2 = bigi
            for e in range(E):
                e_s = jnp.full((LANES,), e, jnp.int32)
                hit = jnp.logical_and(p[e] == m2, e_s != i1)
                i2 = jnp.minimum(i2, jnp.where(hit, e_s, bigi))
            keep = m1 >= thresh
            i1o = jnp.where(keep, i1, neg1i)
            i2o = jnp.where(keep, i2, neg1i)
            w1_v[pl.ds(off, LANES)] = m1
            w2_v[pl.ds(off, LANES)] = m2
            i1_v[pl.ds(off, LANES)] = i1o
            i2_v[pl.ds(off, LANES)] = i2o
            return carry

        lax.fori_loop(0, groups, group, 0)
        pltpu.sync_copy(w1_v, w1_hbm.at[pl.ds(base, per_w)])
        pltpu.sync_copy(w2_v, w2_hbm.at[pl.ds(base, per_w)])
        pltpu.sync_copy(i1_v, i1_hbm.at[pl.ds(base, per_w)])
        pltpu.sync_copy(i2_v, i2_hbm.at[pl.ds(base, per_w)])

    return sc_kernel(probs_t)


NUM_CHUNKS = 2


def kernel(x, W_g):
    B, S, D = x.shape
    N = B * S
    x2 = x.reshape(N, D)
    cn = N // NUM_CHUNKS
    parts = []
    for c in range(NUM_CHUNKS):
        probs_t = _tc_probs(x2[c * cn:(c + 1) * cn], W_g)
        parts.append(_sc_topk_mask(probs_t))
    w1, w2, i1, i2 = (jnp.concatenate([p[j] for p in parts]) for j in range(4))
    wts = jnp.stack([w1, w2], axis=-1).reshape(B, S, TOP_K)
    idx = jnp.stack([i1, i2], axis=-1).reshape(B, S, TOP_K)
    return wts, idx


# speedup vs baseline: 3.7514x; 3.7514x over previous
import jax
import jax.numpy as jnp
from jax import lax
from jax.experimental import pallas as pl

E = 16
TOP_K = 2
CONF_THRESH = 0.7
TILE = 1024


def _router_body(x_ref, w_ref, wts_ref, idx_ref):
    logits = jnp.dot(x_ref[...], w_ref[...], preferred_element_type=jnp.float32)
    lt = logits.T
    m = jnp.max(lt, axis=0, keepdims=True)
    e = jnp.exp(lt - m)
    z = jnp.sum(e, axis=0, keepdims=True)
    p = e / z

    eidx = lax.broadcasted_iota(jnp.int32, p.shape, 0)
    big = jnp.full(p.shape, E, jnp.int32)

    m1 = jnp.max(p, axis=0, keepdims=True)
    i1 = jnp.min(jnp.where(p == m1, eidx, big), axis=0, keepdims=True)
    p2 = jnp.where(eidx == i1, -1.0, p)
    m2 = jnp.max(p2, axis=0, keepdims=True)
    hit2 = jnp.logical_and(p == m2, eidx != i1)
    i2 = jnp.min(jnp.where(hit2, eidx, big), axis=0, keepdims=True)

    keep = m1 >= CONF_THRESH
    i1 = jnp.where(keep, i1, -1)
    i2 = jnp.where(keep, i2, -1)

    wts_ref[...] = jnp.concatenate([m1, m2], axis=0)
    idx_ref[...] = jnp.concatenate([i1, i2], axis=0)


def kernel(x, W_g):
    B, S, D = x.shape
    N = B * S
    x2 = x.reshape(N, D)
    grid = (N // TILE,)
    wts_t, idx_t = pl.pallas_call(
        _router_body,
        grid=grid,
        in_specs=[
            pl.BlockSpec((TILE, D), lambda i: (i, 0)),
            pl.BlockSpec((D, E), lambda i: (0, 0)),
        ],
        out_specs=[
            pl.BlockSpec((TOP_K, TILE), lambda i: (0, i)),
            pl.BlockSpec((TOP_K, TILE), lambda i: (0, i)),
        ],
        out_shape=[
            jax.ShapeDtypeStruct((TOP_K, N), jnp.float32),
            jax.ShapeDtypeStruct((TOP_K, N), jnp.int32),
        ],
    )(x2, W_g)
    wts = wts_t.T.reshape(B, S, TOP_K)
    idx = idx_t.T.reshape(B, S, TOP_K)
    return wts, idx
